# baseline (speedup 1.0000x reference)
import jax
import jax.numpy as jnp
from jax import lax
from jax.experimental import pallas as pl
from jax.experimental.pallas import tpu as pltpu
from jax.experimental.pallas import tpu_sc as plsc

jax.config.update("jax_enable_x64", True)

_L = 16

_MESH = plsc.VectorSubcoreMesh(core_axis_name="c", subcore_axis_name="s",
                               num_cores=1, num_subcores=1)


def _sc_body(in_hbm, out_hbm, in_v, out_v, sem):
    pltpu.async_copy(in_hbm, in_v.at[pl.ds(0, 3)], sem).wait()

    v = in_v[...]
    src_f = v[0]
    ten_f = v[1]
    idx = lax.bitcast_convert_type(v[2], jnp.int32)

    out_f = ten_f * jnp.float32(0.5) + jnp.where(
        idx == 0, src_f, jnp.float32(0.0))

    lanes = lax.iota(jnp.int32, _L)
    out_v[...] = jnp.where(lanes == 0, out_f, jnp.float32(0.0))
    pltpu.sync_copy(out_v.at[pl.ds(0, 1)], out_hbm)


def _scatter_add_halve(packed):
    run = pl.kernel(
        _sc_body,
        out_type=jax.ShapeDtypeStruct((1,), jnp.float32),
        mesh=_MESH,
        scratch_types=[
            pltpu.VMEM((_L,), jnp.float32),
            pltpu.VMEM((_L,), jnp.float32),
            pltpu.SemaphoreType.DMA,
        ],
    )
    return run(packed)


def kernel(source, tensor, index):
    idx_f = lax.bitcast_convert_type(index.astype(jnp.uint32), jnp.float32)
    packed = jnp.concatenate(
        [source.astype(jnp.float32), tensor.astype(jnp.float32), idx_f])
    out = _scatter_add_halve(packed).astype(jnp.float64)
    return (source, out)


# speedup vs baseline: 1.1051x; 1.1051x over previous
import jax
import jax.numpy as jnp
from jax import lax
from jax.experimental import pallas as pl
from jax.experimental.pallas import tpu as pltpu
from jax.experimental.pallas import tpu_sc as plsc

jax.config.update("jax_enable_x64", True)

_L = 16

_MESH = plsc.ScalarSubcoreMesh(axis_name="c", num_cores=1)


def _sc_body(src_hbm, ten_hbm, idx_hbm, out_hbm,
             src_s, ten_s, idx_s, out_s, sem0, sem1, sem2):
    c1 = pltpu.async_copy(src_hbm, src_s, sem0)
    c2 = pltpu.async_copy(ten_hbm, ten_s, sem1)
    c3 = pltpu.async_copy(idx_hbm, idx_s, sem2)
    c1.wait()
    c2.wait()
    c3.wait()

    src_f = src_s[0]
    ten_f = ten_s[0]
    idx = idx_s[0]

    out_f = ten_f * jnp.float32(0.5) + jnp.where(
        idx == 0, src_f, jnp.float32(0.0))

    out_s[0] = out_f
    pltpu.sync_copy(out_s, out_hbm)


def _scatter_add_halve(src32, ten32, idx32):
    run = pl.kernel(
        _sc_body,
        out_type=jax.ShapeDtypeStruct((1,), jnp.float32),
        mesh=_MESH,
        compiler_params=pltpu.CompilerParams(skip_device_barrier=True),
        scratch_types=[
            pltpu.SMEM((1,), jnp.float32),
            pltpu.SMEM((1,), jnp.float32),
            pltpu.SMEM((1,), jnp.uint32),
            pltpu.SMEM((1,), jnp.float32),
            pltpu.SemaphoreType.DMA,
            pltpu.SemaphoreType.DMA,
            pltpu.SemaphoreType.DMA,
        ],
    )
    return run(src32, ten32, idx32)


def kernel(source, tensor, index):
    src32 = source.astype(jnp.float32)
    ten32 = tensor.astype(jnp.float32)
    idx32 = index.astype(jnp.uint32)
    out = _scatter_add_halve(src32, ten32, idx32).astype(jnp.float64)
    return (source, out)
